# baseline (speedup 1.0000x reference)
import functools

import jax
import jax.numpy as jnp
from jax import lax
from jax.experimental import pallas as pl
from jax.experimental.pallas import tpu as pltpu
from jax.experimental.pallas import tpu_sc as plsc

K = 2
CAP_FACTOR = 1.25
_NC = 2
_NS = 16
_NW = _NC * _NS


def _sc_gather(table, idx, chunk):
    n_rows, width = table.shape
    m = idx.shape[0]
    per_w = m // _NW
    assert per_w % chunk == 0 and m % _NW == 0

    mesh = plsc.VectorSubcoreMesh(core_axis_name="c", subcore_axis_name="s")

    @functools.partial(
        pl.kernel,
        mesh=mesh,
        out_type=jax.ShapeDtypeStruct((m, width), table.dtype),
        scratch_types=[
            pltpu.VMEM((chunk,), jnp.int32),
            pltpu.VMEM((chunk, width), table.dtype),
            pltpu.SemaphoreType.DMA,
        ],
    )
    def gather_kernel(table_hbm, idx_hbm, out_hbm, idx_v, rows_v, sem):
        wid = lax.axis_index("s") * _NC + lax.axis_index("c")
        base = wid * per_w

        @pl.loop(0, per_w, step=chunk)
        def _(off):
            pltpu.sync_copy(idx_hbm.at[pl.ds(base + off, chunk)], idx_v)
            pltpu.async_copy(table_hbm.at[idx_v], rows_v, sem).wait()
            pltpu.sync_copy(rows_v, out_hbm.at[pl.ds(base + off, chunk)])

    return gather_kernel(table, idx)


def _ffn_body(x_ref, w1_ref, b1_ref, w2_ref, b2_ref, o_ref):
    hb = pl.program_id(1)
    xb = x_ref[...].astype(jnp.bfloat16)
    w1 = w1_ref[0].astype(jnp.bfloat16)
    h = jnp.dot(xb, w1, preferred_element_type=jnp.float32)
    h = jnp.maximum(h + b1_ref[0], 0.0).astype(jnp.bfloat16)
    part = jnp.dot(h, w2_ref[0].astype(jnp.bfloat16),
                   preferred_element_type=jnp.float32)

    @pl.when(hb == 0)
    def _():
        o_ref[...] = part + b2_ref[0]

    @pl.when(hb != 0)
    def _():
        o_ref[...] = o_ref[...] + part


def _ffn(xg, W1, b1, W2, b2, rows_per_e, hblk):
    e, c, h = W1.shape
    grid = (e, h // hblk)
    return pl.pallas_call(
        _ffn_body,
        grid=grid,
        in_specs=[
            pl.BlockSpec((rows_per_e, c), lambda i, j: (i, 0)),
            pl.BlockSpec((1, c, hblk), lambda i, j: (i, 0, j)),
            pl.BlockSpec((1, 1, hblk), lambda i, j: (i, 0, j)),
            pl.BlockSpec((1, hblk, c), lambda i, j: (i, j, 0)),
            pl.BlockSpec((1, 1, c), lambda i, j: (i, 0, 0)),
        ],
        out_specs=pl.BlockSpec((rows_per_e, c), lambda i, j: (i, 0)),
        out_shape=jax.ShapeDtypeStruct((e * rows_per_e, c), jnp.float32),
    )(xg, W1, b1.reshape(e, 1, h), W2, b2.reshape(e, 1, c))


def _combine_body(a_ref, b_ref, g0_ref, g1_ref, o_ref):
    o_ref[...] = a_ref[...] * g0_ref[...] + b_ref[...] * g1_ref[...]


def _combine(ab, g0, g1, bt, rblk):
    c = ab.shape[1]
    nblk = bt // rblk
    return pl.pallas_call(
        _combine_body,
        grid=(nblk,),
        in_specs=[
            pl.BlockSpec((rblk, c), lambda i: (i, 0)),
            pl.BlockSpec((rblk, c), lambda i: (i + nblk, 0)),
            pl.BlockSpec((rblk, 1), lambda i: (i, 0)),
            pl.BlockSpec((rblk, 1), lambda i: (i, 0)),
        ],
        out_specs=pl.BlockSpec((rblk, c), lambda i: (i, 0)),
        out_shape=jax.ShapeDtypeStruct((bt, c), jnp.float32),
    )(ab, ab, g0, g1)


def kernel(x, Wr, br, W1, b1, W2, b2):
    B, T, C = x.shape
    E = Wr.shape[1]
    H = W1.shape[2]
    cap = int(T / E * CAP_FACTOR)
    S = T * K
    BT = B * T
    R = E * B * cap

    logits = jnp.einsum('btc,ce->bte', x, Wr) + br
    probs = jax.nn.softmax(logits, axis=-1)
    topk_probs, topk_idx = jax.lax.top_k(probs, K)

    eidx = topk_idx.reshape(B, S).astype(jnp.int32)
    oh = jax.nn.one_hot(eidx, E, dtype=jnp.int32)
    cum = jnp.cumsum(oh, axis=1)
    pos = jnp.take_along_axis(cum, eidx[..., None], axis=2)[..., 0] - 1
    valid = pos < cap
    b_ids = jnp.arange(B, dtype=jnp.int32)[:, None]
    yrow = (eidx * B + b_ids) * cap + pos

    src_row = b_ids * T + (jnp.arange(S, dtype=jnp.int32)[None, :] // K)
    flat_yrow = jnp.where(valid, yrow, R).reshape(-1)
    t_src = jnp.zeros((R,), jnp.int32).at[flat_yrow].set(
        src_row.reshape(-1), mode='drop')

    valid3 = valid.reshape(B, T, K)
    rows_k = jnp.where(valid3, yrow.reshape(B, T, K), 0)
    gates = jnp.where(valid3, topk_probs, 0.0)
    comb_idx = jnp.concatenate(
        [rows_k[:, :, 0].reshape(-1), rows_k[:, :, 1].reshape(-1)])
    g0 = gates[:, :, 0].reshape(-1, 1)
    g1 = gates[:, :, 1].reshape(-1, 1)

    x_flat = x.reshape(BT, C)
    xg = _sc_gather(x_flat, t_src, chunk=80)
    y = _ffn(xg, W1, b1, W2, b2, rows_per_e=B * cap, hblk=1024)
    ab = _sc_gather(y, comb_idx, chunk=64)
    out = _combine(ab, g0, g1, bt=BT, rblk=512)
    return out.reshape(B, T, C)
